# baseline (speedup 1.0000x reference)
import functools

import jax
import jax.numpy as jnp
from jax import lax
from jax.experimental import pallas as pl
from jax.experimental.pallas import tpu as pltpu
from jax.experimental.pallas import tpu_sc as plsc

BATCH = 16384
FACTOR = 32
PACK = 128 // FACTOR
SROWS = 1000000 // PACK

NUM_CORES = 2
NUM_SUBCORES = 16
NUM_WORKERS = NUM_CORES * NUM_SUBCORES
BPW = BATCH // NUM_WORKERS
CHUNK = 128
HALF = BPW // 2
NCH = HALF // CHUNK
LANES = 16


def _sc_gather_body(uidx_hbm, iidx_hbm, utab_hbm, itab_hbm,
                    uout_hbm, iout_hbm,
                    idx_v, buf_a, buf_b, sem):
  wid = lax.axis_index("s") * NUM_CORES + lax.axis_index("c")
  base = wid * BPW
  nch_tab = BPW // CHUNK
  for j in range(nch_tab):
    pltpu.sync_copy(uidx_hbm.at[pl.ds(base + j * CHUNK, CHUNK)],
                    idx_v.at[j])
    pltpu.sync_copy(iidx_hbm.at[pl.ds(base + j * CHUNK, CHUNK)],
                    idx_v.at[nch_tab + j])
  for j in range(2 * nch_tab):
    for v in range(CHUNK // LANES):
      sl = pl.ds(v * LANES, LANES)
      idx_v[j, sl] = lax.shift_right_logical(idx_v[j, sl], 2)

  def gather(tab, half_idx, buf):
    cps = []
    for j in range(NCH):
      cps.append(pltpu.async_copy(
          tab.at[idx_v.at[half_idx * NCH + j]],
          buf.at[pl.ds(j * CHUNK, CHUNK)], sem))
    return cps

  def drain(cps):
    for c in cps:
      c.wait()

  cps = gather(utab_hbm, 0, buf_a)
  drain(cps)
  cps = gather(utab_hbm, 1, buf_b)
  pltpu.sync_copy(buf_a, uout_hbm.at[pl.ds(base, HALF)])
  drain(cps)
  cps = gather(itab_hbm, 2, buf_a)
  pltpu.sync_copy(buf_b, uout_hbm.at[pl.ds(base + HALF, HALF)])
  drain(cps)
  cps = gather(itab_hbm, 3, buf_b)
  pltpu.sync_copy(buf_a, iout_hbm.at[pl.ds(base, HALF)])
  drain(cps)
  pltpu.sync_copy(buf_b, iout_hbm.at[pl.ds(base + HALF, HALF)])


_sc_gather = functools.partial(
    pl.kernel,
    out_type=(
        jax.ShapeDtypeStruct((BATCH, 128), jnp.float32),
        jax.ShapeDtypeStruct((BATCH, 128), jnp.float32),
    ),
    mesh=plsc.VectorSubcoreMesh(core_axis_name="c", subcore_axis_name="s"),
    scratch_types=[
        pltpu.VMEM((2 * (BPW // CHUNK), CHUNK), jnp.int32),
        pltpu.VMEM((HALF, 128), jnp.float32),
        pltpu.VMEM((HALF, 128), jnp.float32),
        pltpu.SemaphoreType.DMA,
    ],
)(_sc_gather_body)


TC_BLK = 512


def _tc_affine_body(u_ref, i_ref, uidx_ref, iidx_ref, w_ref, b_ref, o_ref):
  qu = uidx_ref[...] & (PACK - 1)
  qi = iidx_ref[...] & (PACK - 1)
  u128 = u_ref[...]
  i128 = i_ref[...]
  uq = jnp.zeros((TC_BLK, FACTOR), jnp.float32)
  iq = jnp.zeros((TC_BLK, FACTOR), jnp.float32)
  for q in range(PACK):
    sl = slice(q * FACTOR, (q + 1) * FACTOR)
    uq = uq + jnp.where(qu == q, u128[:, sl], 0.0)
    iq = iq + jnp.where(qi == q, i128[:, sl], 0.0)
  s = jnp.sum(uq * iq * w_ref[...], axis=1) + b_ref[0, 0]
  o_ref[...] = jax.nn.sigmoid(s)


def _tc_affine(u_rows, i_rows, uidx, iidx, affine_w, affine_b):
  grid = (BATCH // TC_BLK,)
  return pl.pallas_call(
      _tc_affine_body,
      grid=grid,
      in_specs=[
          pl.BlockSpec((TC_BLK, 128), lambda i: (i, 0)),
          pl.BlockSpec((TC_BLK, 128), lambda i: (i, 0)),
          pl.BlockSpec((TC_BLK, 1), lambda i: (i, 0)),
          pl.BlockSpec((TC_BLK, 1), lambda i: (i, 0)),
          pl.BlockSpec((1, FACTOR), lambda i: (0, 0)),
          pl.BlockSpec(memory_space=pltpu.SMEM),
      ],
      out_specs=pl.BlockSpec((TC_BLK,), lambda i: (i,)),
      out_shape=jax.ShapeDtypeStruct((BATCH,), jnp.float32),
  )(u_rows, i_rows, uidx.reshape(BATCH, 1), iidx.reshape(BATCH, 1),
    affine_w, affine_b.reshape(1, 1))


def kernel(user_indices, item_indices, user_table, item_table,
           affine_w, affine_b):
  uidx = user_indices.astype(jnp.int32)
  iidx = item_indices.astype(jnp.int32)
  utab = user_table.reshape(SROWS, 128)
  itab = item_table.reshape(SROWS, 128)
  u_rows, i_rows = _sc_gather(uidx, iidx, utab, itab)
  return _tc_affine(u_rows, i_rows, uidx, iidx, affine_w, affine_b)
